# baseline (speedup 1.0000x reference)
import functools

import jax
import jax.numpy as jnp
from jax import lax
from jax.experimental import pallas as pl
from jax.experimental.pallas import tpu as pltpu
from jax.experimental.pallas import tpu_sc as plsc

_N = 10000
_D = 256
_E_EXP = 8
_C = 64

_L = 16
_NS = 16
_NC = 2
_K = 128
_EBLK = 4096
_HALF = 5120
_DUMMY = _HALF
_ACC_ROWS = _HALF + 8


def _sc_agg_kernel(h16_hbm, src_hbm, dst_hbm, ew_hbm, out_hbm,
                   srcb_v, dstb_v, ewb_v, gidx0_v, gidx1_v,
                   rows0_v, rows1_v, acc_v, sem0, sem1, *, nblk):
    c = lax.axis_index("c")
    s = lax.axis_index("s")
    lo = c * _HALF
    s16 = lax.broadcast(s, (_L,))

    def _zero(i, _):
        acc_v[i] = jnp.zeros((_L,), jnp.float32)
        return 0
    lax.fori_loop(0, _ACC_ROWS, _zero, 0)

    def _gidx(cb, gidx_v):
        for j in range(_K // _L):
            sv = srcb_v[pl.ds(cb + j * _L, _L)]
            gidx_v[pl.ds(j * _L, _L)] = sv * _L + s16

    def _fire(gidx_v, rows_v, sem):
        pltpu.async_copy(h16_hbm.at[gidx_v], rows_v, sem)

    def _wait(gidx_v, rows_v, sem):
        pltpu.make_async_copy(h16_hbm.at[gidx_v], rows_v, sem).wait()

    def _accum(cb, rows_v):
        for j in range(_K // _L):
            d16 = dstb_v[pl.ds(cb + j * _L, _L)]
            w16 = ewb_v[pl.ds(cb + j * _L, _L)]
            dloc = d16 - lo
            valid = (d16 >= lo) & (dloc < _HALF)
            didx = jnp.where(valid, dloc, _DUMMY)
            for l in range(_L):
                il = didx[l]
                wv = lax.broadcast(w16[l], (_L,))
                plsc.addupdate(acc_v.at[il], rows_v[j * _L + l] * wv)

    def _block(b, _):
        ebase = b * _EBLK
        pltpu.sync_copy(src_hbm.at[pl.ds(ebase, _EBLK)], srcb_v)
        pltpu.sync_copy(dst_hbm.at[pl.ds(ebase, _EBLK)], dstb_v)
        pltpu.sync_copy(ew_hbm.at[pl.ds(ebase, _EBLK)], ewb_v)
        _gidx(0, gidx0_v)
        _fire(gidx0_v, rows0_v, sem0)

        def _pair(k, _):
            cb0 = (2 * k) * _K
            cb1 = (2 * k + 1) * _K
            _gidx(cb1, gidx1_v)
            _fire(gidx1_v, rows1_v, sem1)
            _wait(gidx0_v, rows0_v, sem0)
            _accum(cb0, rows0_v)
            @pl.when(k < (_EBLK // (2 * _K)) - 1)
            def _():
                _gidx(cb1 + _K, gidx0_v)
                _fire(gidx0_v, rows0_v, sem0)
            _wait(gidx1_v, rows1_v, sem1)
            _accum(cb1, rows1_v)
            return 0
        lax.fori_loop(0, _EBLK // (2 * _K), _pair, 0)
        return 0
    lax.fori_loop(0, nblk, _block, 0)

    pltpu.sync_copy(acc_v.at[pl.ds(0, _HALF)],
                    out_hbm.at[s].at[pl.ds(c * _HALF, _HALF)])


def _sc_agg(h, src, dst, ew):
    e = src.shape[0]
    epad = ((e + _EBLK - 1) // _EBLK) * _EBLK
    nblk = epad // _EBLK
    if epad != e:
        pad = epad - e
        src = jnp.concatenate([src, jnp.zeros((pad,), jnp.int32)])
        dst = jnp.concatenate([dst, jnp.full((pad,), _NC * _HALF, jnp.int32)])
        ew = jnp.concatenate([ew, jnp.zeros((pad,), jnp.float32)])
    h16 = h.reshape(_N * _L, _L)

    mesh = plsc.VectorSubcoreMesh(core_axis_name="c", subcore_axis_name="s")
    run = functools.partial(
        pl.kernel,
        mesh=mesh,
        compiler_params=pltpu.CompilerParams(use_tc_tiling_on_sc=False),
        out_type=jax.ShapeDtypeStruct((_NS, _NC * _HALF, _L), jnp.float32),
        scratch_types=[
            pltpu.VMEM((_EBLK,), jnp.int32),
            pltpu.VMEM((_EBLK,), jnp.int32),
            pltpu.VMEM((_EBLK,), jnp.float32),
            pltpu.VMEM((_K,), jnp.int32),
            pltpu.VMEM((_K,), jnp.int32),
            pltpu.VMEM((_K, _L), jnp.float32),
            pltpu.VMEM((_K, _L), jnp.float32),
            pltpu.VMEM((_ACC_ROWS, _L), jnp.float32),
            pltpu.SemaphoreType.DMA,
            pltpu.SemaphoreType.DMA,
        ],
    )(functools.partial(_sc_agg_kernel, nblk=nblk))
    out = run(h16, src, dst, ew)
    agg = out.transpose(1, 0, 2).reshape(_NC * _HALF, _D)
    return agg[:_N]


def _top2_dense(lg):
    ids = lax.broadcasted_iota(jnp.int32, lg.shape, 1)
    ne = lg.shape[1]
    v1 = jnp.max(lg, axis=1, keepdims=True)
    i1 = jnp.min(jnp.where(lg == v1, ids, ne), axis=1, keepdims=True)
    masked = jnp.where(ids == i1, -jnp.inf, lg)
    v2 = jnp.max(masked, axis=1, keepdims=True)
    i2 = jnp.min(jnp.where(masked == v2, ids, ne), axis=1, keepdims=True)
    t = jnp.exp(v2 - v1)
    den = 1.0 + t
    gates = jnp.where(ids == i1, 1.0 / den, 0.0) + jnp.where(ids == i2, t / den, 0.0)
    return gates, i1, i2, ids


def _tc_moe_kernel(h_ref, agg_ref, wf_ref, bf_ref, wg1_ref, wg2_ref,
                   wc_ref, bc_ref, out_ref, gates_ref, idx_ref):
    agg_blk = agg_ref[...]
    eo = jnp.dot(agg_blk, wf_ref[...], preferred_element_type=jnp.float32) + bf_ref[...]
    lg1 = jnp.dot(h_ref[...], wg1_ref[...], preferred_element_type=jnp.float32)
    g1, _, _, _ = _top2_dense(lg1)
    h_moe = g1[:, 0:1] * eo[:, 0:_D]
    for e in range(1, _E_EXP):
        h_moe = h_moe + g1[:, e:e + 1] * eo[:, e * _D:(e + 1) * _D]
    lg2 = jnp.dot(h_moe, wg2_ref[...], preferred_element_type=jnp.float32)
    g2, i1, i2, ids = _top2_dense(lg2)
    h_new = g2[:, 0:1] * eo[:, 0:_D]
    for e in range(1, _E_EXP):
        h_new = h_new + g2[:, e:e + 1] * eo[:, e * _D:(e + 1) * _D]
    out_ref[...] = jnp.dot(h_new, wc_ref[...], preferred_element_type=jnp.float32) + bc_ref[...]
    gates_ref[...] = g2
    idx_ref[...] = jnp.where(ids == 0, i1, jnp.where(ids == 1, i2, 0))


def _tc_moe(h, agg, w_gate_moe, W_e, b_e, w_gate_pur, W_c, b_c):
    bn = 400
    grid = (_N // bn,)
    wf = W_e.transpose(1, 0, 2).reshape(_D, _E_EXP * _D)
    bf = b_e.reshape(1, _E_EXP * _D)
    bc = b_c.reshape(1, _C)
    blk = lambda shape: pl.BlockSpec(shape, lambda i: (0, 0))
    out = pl.pallas_call(
        _tc_moe_kernel,
        grid=grid,
        in_specs=[
            pl.BlockSpec((bn, _D), lambda i: (i, 0)),
            pl.BlockSpec((bn, _D), lambda i: (i, 0)),
            blk((_D, _E_EXP * _D)),
            blk((1, _E_EXP * _D)),
            blk((_D, _E_EXP)),
            blk((_D, _E_EXP)),
            blk((_D, _C)),
            blk((1, _C)),
        ],
        out_specs=[
            pl.BlockSpec((bn, _C), lambda i: (i, 0)),
            pl.BlockSpec((bn, _E_EXP), lambda i: (i, 0)),
            pl.BlockSpec((bn, _E_EXP), lambda i: (i, 0)),
        ],
        out_shape=[
            jax.ShapeDtypeStruct((_N, _C), jnp.float32),
            jax.ShapeDtypeStruct((_N, _E_EXP), jnp.float32),
            jax.ShapeDtypeStruct((_N, _E_EXP), jnp.int32),
        ],
    )(h, agg, wf, bf, w_gate_moe, w_gate_pur, W_c, bc)
    logits, gates, idx8 = out
    return logits, gates, idx8[:, :2]


def kernel(h, edge_index, edge_weight, w_gate_moe, W_e, b_e, w_gate_pur, W_c, b_c):
    src = edge_index[0]
    dst = edge_index[1]
    agg = _sc_agg(h, src, dst, edge_weight)
    return _tc_moe(h, agg, w_gate_moe, W_e, b_e, w_gate_pur, W_c, b_c)


# speedup vs baseline: 1.0342x; 1.0342x over previous
import functools

import jax
import jax.numpy as jnp
from jax import lax
from jax.experimental import pallas as pl
from jax.experimental.pallas import tpu as pltpu
from jax.experimental.pallas import tpu_sc as plsc

_N = 10000
_D = 256
_E_EXP = 8
_C = 64

_L = 16
_NS = 16
_NC = 2
_K = 128
_EBLK = 4096
_HALF = 5120
_DUMMY = _HALF
_ACC_ROWS = _HALF + 8


def _sc_agg_kernel(h16_hbm, src_hbm, dst_hbm, ew_hbm, out_hbm,
                   srcb_v, dstb_v, ewb_v, gidx0_v, gidx1_v,
                   rows0_v, rows1_v, acc_v, sem0, sem1, *, nblk):
    c = lax.axis_index("c")
    s = lax.axis_index("s")
    lo = c * _HALF
    s16 = lax.broadcast(s, (_L,))

    def _zero(i, _):
        acc_v[i] = jnp.zeros((_L,), jnp.float32)
        return 0
    lax.fori_loop(0, _ACC_ROWS, _zero, 0)

    def _gidx(cb, gidx_v):
        for j in range(_K // _L):
            sv = srcb_v[pl.ds(cb + j * _L, _L)]
            gidx_v[pl.ds(j * _L, _L)] = sv * _L + s16

    def _fire(gidx_v, rows_v, sem):
        pltpu.async_copy(h16_hbm.at[gidx_v], rows_v, sem)

    def _wait(gidx_v, rows_v, sem):
        pltpu.make_async_copy(h16_hbm.at[gidx_v], rows_v, sem).wait()

    def _accum(cb, rows_v):
        for j in range(_K // _L):
            d16 = dstb_v[pl.ds(cb + j * _L, _L)]
            w16 = ewb_v[pl.ds(cb + j * _L, _L)]
            dloc = d16 - lo
            valid = (d16 >= lo) & (dloc < _HALF)
            didx = jnp.where(valid, dloc, _DUMMY)
            for l in range(_L):
                il = didx[l]
                wv = lax.broadcast(w16[l], (_L,))
                plsc.addupdate(acc_v.at[il], rows_v[j * _L + l] * wv)

    bofs = ((s * _NC + c) * nblk) // (_NS * _NC)

    def _block(b0, _):
        b = lax.rem(b0 + bofs, nblk)
        ebase = b * _EBLK
        pltpu.sync_copy(src_hbm.at[pl.ds(ebase, _EBLK)], srcb_v)
        pltpu.sync_copy(dst_hbm.at[pl.ds(ebase, _EBLK)], dstb_v)
        pltpu.sync_copy(ew_hbm.at[pl.ds(ebase, _EBLK)], ewb_v)
        _gidx(0, gidx0_v)
        _fire(gidx0_v, rows0_v, sem0)

        def _pair(k, _):
            cb0 = (2 * k) * _K
            cb1 = (2 * k + 1) * _K
            _gidx(cb1, gidx1_v)
            _fire(gidx1_v, rows1_v, sem1)
            _wait(gidx0_v, rows0_v, sem0)
            _accum(cb0, rows0_v)
            @pl.when(k < (_EBLK // (2 * _K)) - 1)
            def _():
                _gidx(cb1 + _K, gidx0_v)
                _fire(gidx0_v, rows0_v, sem0)
            _wait(gidx1_v, rows1_v, sem1)
            _accum(cb1, rows1_v)
            return 0
        lax.fori_loop(0, _EBLK // (2 * _K), _pair, 0)
        return 0
    lax.fori_loop(0, nblk, _block, 0)

    pltpu.sync_copy(acc_v.at[pl.ds(0, _HALF)],
                    out_hbm.at[s].at[pl.ds(c * _HALF, _HALF)])


def _sc_agg(h, src, dst, ew):
    e = src.shape[0]
    epad = ((e + _EBLK - 1) // _EBLK) * _EBLK
    nblk = epad // _EBLK
    if epad != e:
        pad = epad - e
        src = jnp.concatenate([src, jnp.zeros((pad,), jnp.int32)])
        dst = jnp.concatenate([dst, jnp.full((pad,), _NC * _HALF, jnp.int32)])
        ew = jnp.concatenate([ew, jnp.zeros((pad,), jnp.float32)])
    h16 = h.reshape(_N * _L, _L)

    mesh = plsc.VectorSubcoreMesh(core_axis_name="c", subcore_axis_name="s")
    run = functools.partial(
        pl.kernel,
        mesh=mesh,
        compiler_params=pltpu.CompilerParams(use_tc_tiling_on_sc=False),
        out_type=jax.ShapeDtypeStruct((_NS, _NC * _HALF, _L), jnp.float32),
        scratch_types=[
            pltpu.VMEM((_EBLK,), jnp.int32),
            pltpu.VMEM((_EBLK,), jnp.int32),
            pltpu.VMEM((_EBLK,), jnp.float32),
            pltpu.VMEM((_K,), jnp.int32),
            pltpu.VMEM((_K,), jnp.int32),
            pltpu.VMEM((_K, _L), jnp.float32),
            pltpu.VMEM((_K, _L), jnp.float32),
            pltpu.VMEM((_ACC_ROWS, _L), jnp.float32),
            pltpu.SemaphoreType.DMA,
            pltpu.SemaphoreType.DMA,
        ],
    )(functools.partial(_sc_agg_kernel, nblk=nblk))
    out = run(h16, src, dst, ew)
    agg = out.transpose(1, 0, 2).reshape(_NC * _HALF, _D)
    return agg[:_N]


def _top2_dense(lg):
    ids = lax.broadcasted_iota(jnp.int32, lg.shape, 1)
    ne = lg.shape[1]
    v1 = jnp.max(lg, axis=1, keepdims=True)
    i1 = jnp.min(jnp.where(lg == v1, ids, ne), axis=1, keepdims=True)
    masked = jnp.where(ids == i1, -jnp.inf, lg)
    v2 = jnp.max(masked, axis=1, keepdims=True)
    i2 = jnp.min(jnp.where(masked == v2, ids, ne), axis=1, keepdims=True)
    t = jnp.exp(v2 - v1)
    den = 1.0 + t
    gates = jnp.where(ids == i1, 1.0 / den, 0.0) + jnp.where(ids == i2, t / den, 0.0)
    return gates, i1, i2, ids


def _tc_moe_kernel(h_ref, agg_ref, wf_ref, bf_ref, wg1_ref, wg2_ref,
                   wc_ref, bc_ref, out_ref, gates_ref, idx_ref):
    agg_blk = agg_ref[...]
    eo = jnp.dot(agg_blk, wf_ref[...], preferred_element_type=jnp.float32) + bf_ref[...]
    lg1 = jnp.dot(h_ref[...], wg1_ref[...], preferred_element_type=jnp.float32)
    g1, _, _, _ = _top2_dense(lg1)
    h_moe = g1[:, 0:1] * eo[:, 0:_D]
    for e in range(1, _E_EXP):
        h_moe = h_moe + g1[:, e:e + 1] * eo[:, e * _D:(e + 1) * _D]
    lg2 = jnp.dot(h_moe, wg2_ref[...], preferred_element_type=jnp.float32)
    g2, i1, i2, ids = _top2_dense(lg2)
    h_new = g2[:, 0:1] * eo[:, 0:_D]
    for e in range(1, _E_EXP):
        h_new = h_new + g2[:, e:e + 1] * eo[:, e * _D:(e + 1) * _D]
    out_ref[...] = jnp.dot(h_new, wc_ref[...], preferred_element_type=jnp.float32) + bc_ref[...]
    gates_ref[...] = g2
    idx_ref[...] = jnp.where(ids == 0, i1, jnp.where(ids == 1, i2, 0))


def _tc_moe(h, agg, w_gate_moe, W_e, b_e, w_gate_pur, W_c, b_c):
    bn = 400
    grid = (_N // bn,)
    wf = W_e.transpose(1, 0, 2).reshape(_D, _E_EXP * _D)
    bf = b_e.reshape(1, _E_EXP * _D)
    bc = b_c.reshape(1, _C)
    blk = lambda shape: pl.BlockSpec(shape, lambda i: (0, 0))
    out = pl.pallas_call(
        _tc_moe_kernel,
        grid=grid,
        in_specs=[
            pl.BlockSpec((bn, _D), lambda i: (i, 0)),
            pl.BlockSpec((bn, _D), lambda i: (i, 0)),
            blk((_D, _E_EXP * _D)),
            blk((1, _E_EXP * _D)),
            blk((_D, _E_EXP)),
            blk((_D, _E_EXP)),
            blk((_D, _C)),
            blk((1, _C)),
        ],
        out_specs=[
            pl.BlockSpec((bn, _C), lambda i: (i, 0)),
            pl.BlockSpec((bn, _E_EXP), lambda i: (i, 0)),
            pl.BlockSpec((bn, _E_EXP), lambda i: (i, 0)),
        ],
        out_shape=[
            jax.ShapeDtypeStruct((_N, _C), jnp.float32),
            jax.ShapeDtypeStruct((_N, _E_EXP), jnp.float32),
            jax.ShapeDtypeStruct((_N, _E_EXP), jnp.int32),
        ],
    )(h, agg, wf, bf, w_gate_moe, w_gate_pur, W_c, bc)
    logits, gates, idx8 = out
    return logits, gates, idx8[:, :2]


def kernel(h, edge_index, edge_weight, w_gate_moe, W_e, b_e, w_gate_pur, W_c, b_c):
    src = edge_index[0]
    dst = edge_index[1]
    agg = _sc_agg(h, src, dst, edge_weight)
    return _tc_moe(h, agg, w_gate_moe, W_e, b_e, w_gate_pur, W_c, b_c)


# speedup vs baseline: 1.8053x; 1.7456x over previous
import functools

import jax
import jax.numpy as jnp
from jax import lax
from jax.experimental import pallas as pl
from jax.experimental.pallas import tpu as pltpu
from jax.experimental.pallas import tpu_sc as plsc

_N = 10000
_D = 256
_E_EXP = 8
_C = 64

_L = 16
_NS = 16
_NC = 2
_K = 128
_EBLK = 4096
_HALF = 5120
_DUMMY = _HALF
_ACC_ROWS = _HALF + 8


_NW = _NS * _NC
_SHARE = 5120
_CAP = _SHARE + 2 * _K


def _sc_part_kernel(src_hbm, dst_hbm, ew_hbm,
                    srcbin_hbm, dstbin_hbm, ewbin_hbm, cnt_hbm,
                    srcs_v, dsts_v, ews_v,
                    sa_v, da_v, wa_v, sb_v, db_v, wb_v, cnt_v):
    c = lax.axis_index("c")
    s = lax.axis_index("s")
    sid = c * _NS + s
    ebase = sid * _SHARE
    pltpu.sync_copy(src_hbm.at[pl.ds(ebase, _SHARE)], srcs_v)
    pltpu.sync_copy(dst_hbm.at[pl.ds(ebase, _SHARE)], dsts_v)
    pltpu.sync_copy(ew_hbm.at[pl.ds(ebase, _SHARE)], ews_v)

    def _grp(g, p):
        pa, pb = p
        s16 = srcs_v[pl.ds(g * _L, _L)]
        d16 = dsts_v[pl.ds(g * _L, _L)]
        w16 = ews_v[pl.ds(g * _L, _L)]
        isa = jnp.where(d16 < _HALF, 1, 0)
        dbl = d16 - _HALF
        for l in range(_L):
            sv = lax.broadcast(s16[l], (_L,))
            wv = lax.broadcast(w16[l], (_L,))
            sa_v[pl.ds(pa, _L)] = sv
            da_v[pl.ds(pa, _L)] = lax.broadcast(d16[l], (_L,))
            wa_v[pl.ds(pa, _L)] = wv
            sb_v[pl.ds(pb, _L)] = sv
            db_v[pl.ds(pb, _L)] = lax.broadcast(dbl[l], (_L,))
            wb_v[pl.ds(pb, _L)] = wv
            al = isa[l]
            pa = pa + al
            pb = pb + (1 - al)
        return pa, pb
    pa, pb = lax.fori_loop(0, _SHARE // _L, _grp, (0, 0))

    zsrc = jnp.zeros((_L,), jnp.int32)
    zdum = jnp.full((_L,), _DUMMY, jnp.int32)
    zw = jnp.zeros((_L,), jnp.float32)
    for t in range(2 * _K // _L):
        sa_v[pl.ds(pa + t * _L, _L)] = zsrc
        da_v[pl.ds(pa + t * _L, _L)] = zdum
        wa_v[pl.ds(pa + t * _L, _L)] = zw
        sb_v[pl.ds(pb + t * _L, _L)] = zsrc
        db_v[pl.ds(pb + t * _L, _L)] = zdum
        wb_v[pl.ds(pb + t * _L, _L)] = zw
    npa = (pa + 2 * _K - 1) // (2 * _K)
    npb = (pb + 2 * _K - 1) // (2 * _K)

    pltpu.sync_copy(sa_v, srcbin_hbm.at[0].at[sid])
    pltpu.sync_copy(da_v, dstbin_hbm.at[0].at[sid])
    pltpu.sync_copy(wa_v, ewbin_hbm.at[0].at[sid])
    pltpu.sync_copy(sb_v, srcbin_hbm.at[1].at[sid])
    pltpu.sync_copy(db_v, dstbin_hbm.at[1].at[sid])
    pltpu.sync_copy(wb_v, ewbin_hbm.at[1].at[sid])
    cnt_v[pl.ds(0, _L)] = lax.broadcast(npa, (_L,))
    pltpu.sync_copy(cnt_v, cnt_hbm.at[0].at[sid])
    cnt_v[pl.ds(0, _L)] = lax.broadcast(npb, (_L,))
    pltpu.sync_copy(cnt_v, cnt_hbm.at[1].at[sid])


def _sc_accum_kernel(h16_hbm, srcbin_hbm, dstbin_hbm, ewbin_hbm, cnt_hbm,
                     out_hbm, srcs_v, dsts_v, ews_v, cnt_v,
                     gidx0_v, gidx1_v, rows0_v, rows1_v, acc_v, sem0, sem1):
    c = lax.axis_index("c")
    s = lax.axis_index("s")
    s16 = lax.broadcast(s, (_L,))

    def _zero(i, _):
        acc_v[i] = jnp.zeros((_L,), jnp.float32)
        return 0
    lax.fori_loop(0, _ACC_ROWS, _zero, 0)

    def _gidx(cb, gidx_v):
        for j in range(_K // _L):
            sv = srcs_v[pl.ds(cb + j * _L, _L)]
            gidx_v[pl.ds(j * _L, _L)] = sv * _L + s16

    def _accum(cb, rows_v):
        for j in range(_K // _L):
            didx = dsts_v[pl.ds(cb + j * _L, _L)]
            w16 = ews_v[pl.ds(cb + j * _L, _L)]
            for l in range(_L):
                il = didx[l]
                wv = lax.broadcast(w16[l], (_L,))
                plsc.addupdate(acc_v.at[il], rows_v[j * _L + l] * wv)

    def _share(i0, _):
        i = lax.rem(i0 + s * _NC + c, _NW)
        pltpu.sync_copy(srcbin_hbm.at[c].at[i], srcs_v)
        pltpu.sync_copy(dstbin_hbm.at[c].at[i], dsts_v)
        pltpu.sync_copy(ewbin_hbm.at[c].at[i], ews_v)
        pltpu.sync_copy(cnt_hbm.at[c].at[i], cnt_v)
        cntrow = cnt_v[pl.ds(0, _L)]
        npair = cntrow[0]

        @pl.when(npair > 0)
        def _():
            _gidx(0, gidx0_v)
            pltpu.async_copy(h16_hbm.at[gidx0_v], rows0_v, sem0)

            def _pair(k, _):
                cb0 = (2 * k) * _K
                cb1 = (2 * k + 1) * _K
                _gidx(cb1, gidx1_v)
                pltpu.async_copy(h16_hbm.at[gidx1_v], rows1_v, sem1)
                pltpu.make_async_copy(h16_hbm.at[gidx0_v], rows0_v, sem0).wait()
                _accum(cb0, rows0_v)
                @pl.when(k < npair - 1)
                def _():
                    _gidx(cb1 + _K, gidx0_v)
                    pltpu.async_copy(h16_hbm.at[gidx0_v], rows0_v, sem0)
                pltpu.make_async_copy(h16_hbm.at[gidx1_v], rows1_v, sem1).wait()
                _accum(cb1, rows1_v)
                return 0
            lax.fori_loop(0, npair, _pair, 0)
        return 0
    lax.fori_loop(0, _NW, _share, 0)

    pltpu.sync_copy(acc_v.at[pl.ds(0, _HALF)],
                    out_hbm.at[s].at[pl.ds(c * _HALF, _HALF)])


def _sc_agg(h, src, dst, ew):
    e = src.shape[0]
    epad = _NW * _SHARE
    if epad != e:
        pad = epad - e
        src = jnp.concatenate([src, jnp.zeros((pad,), jnp.int32)])
        dst = jnp.concatenate([dst, jnp.full((pad,), _NC * _HALF, jnp.int32)])
        ew = jnp.concatenate([ew, jnp.zeros((pad,), jnp.float32)])
    h16 = h.reshape(_N * _L, _L)

    mesh = plsc.VectorSubcoreMesh(core_axis_name="c", subcore_axis_name="s")
    part = functools.partial(
        pl.kernel,
        mesh=mesh,
        compiler_params=pltpu.CompilerParams(use_tc_tiling_on_sc=False),
        out_type=[
            jax.ShapeDtypeStruct((_NC, _NW, _CAP), jnp.int32),
            jax.ShapeDtypeStruct((_NC, _NW, _CAP), jnp.int32),
            jax.ShapeDtypeStruct((_NC, _NW, _CAP), jnp.float32),
            jax.ShapeDtypeStruct((_NC, _NW, _L), jnp.int32),
        ],
        scratch_types=[
            pltpu.VMEM((_SHARE,), jnp.int32),
            pltpu.VMEM((_SHARE,), jnp.int32),
            pltpu.VMEM((_SHARE,), jnp.float32),
            pltpu.VMEM((_CAP,), jnp.int32),
            pltpu.VMEM((_CAP,), jnp.int32),
            pltpu.VMEM((_CAP,), jnp.float32),
            pltpu.VMEM((_CAP,), jnp.int32),
            pltpu.VMEM((_CAP,), jnp.int32),
            pltpu.VMEM((_CAP,), jnp.float32),
            pltpu.VMEM((_L,), jnp.int32),
        ],
    )(_sc_part_kernel)
    srcbin, dstbin, ewbin, cnt = part(src, dst, ew)

    accum = functools.partial(
        pl.kernel,
        mesh=mesh,
        compiler_params=pltpu.CompilerParams(use_tc_tiling_on_sc=False),
        out_type=jax.ShapeDtypeStruct((_NS, _NC * _HALF, _L), jnp.float32),
        scratch_types=[
            pltpu.VMEM((_CAP,), jnp.int32),
            pltpu.VMEM((_CAP,), jnp.int32),
            pltpu.VMEM((_CAP,), jnp.float32),
            pltpu.VMEM((_L,), jnp.int32),
            pltpu.VMEM((_K,), jnp.int32),
            pltpu.VMEM((_K,), jnp.int32),
            pltpu.VMEM((_K, _L), jnp.float32),
            pltpu.VMEM((_K, _L), jnp.float32),
            pltpu.VMEM((_ACC_ROWS, _L), jnp.float32),
            pltpu.SemaphoreType.DMA,
            pltpu.SemaphoreType.DMA,
        ],
    )(_sc_accum_kernel)
    out = accum(h16, srcbin, dstbin, ewbin, cnt)
    agg = out.transpose(1, 0, 2).reshape(_NC * _HALF, _D)
    return agg[:_N]


def _top2_dense(lg):
    ids = lax.broadcasted_iota(jnp.int32, lg.shape, 1)
    ne = lg.shape[1]
    v1 = jnp.max(lg, axis=1, keepdims=True)
    i1 = jnp.min(jnp.where(lg == v1, ids, ne), axis=1, keepdims=True)
    masked = jnp.where(ids == i1, -jnp.inf, lg)
    v2 = jnp.max(masked, axis=1, keepdims=True)
    i2 = jnp.min(jnp.where(masked == v2, ids, ne), axis=1, keepdims=True)
    t = jnp.exp(v2 - v1)
    den = 1.0 + t
    gates = jnp.where(ids == i1, 1.0 / den, 0.0) + jnp.where(ids == i2, t / den, 0.0)
    return gates, i1, i2, ids


def _tc_moe_kernel(h_ref, agg_ref, wf_ref, bf_ref, wg1_ref, wg2_ref,
                   wc_ref, bc_ref, out_ref, gates_ref, idx_ref):
    agg_blk = agg_ref[...]
    eo = jnp.dot(agg_blk, wf_ref[...], preferred_element_type=jnp.float32) + bf_ref[...]
    lg1 = jnp.dot(h_ref[...], wg1_ref[...], preferred_element_type=jnp.float32)
    g1, _, _, _ = _top2_dense(lg1)
    h_moe = g1[:, 0:1] * eo[:, 0:_D]
    for e in range(1, _E_EXP):
        h_moe = h_moe + g1[:, e:e + 1] * eo[:, e * _D:(e + 1) * _D]
    lg2 = jnp.dot(h_moe, wg2_ref[...], preferred_element_type=jnp.float32)
    g2, i1, i2, ids = _top2_dense(lg2)
    h_new = g2[:, 0:1] * eo[:, 0:_D]
    for e in range(1, _E_EXP):
        h_new = h_new + g2[:, e:e + 1] * eo[:, e * _D:(e + 1) * _D]
    out_ref[...] = jnp.dot(h_new, wc_ref[...], preferred_element_type=jnp.float32) + bc_ref[...]
    gates_ref[...] = g2
    idx_ref[...] = jnp.where(ids == 0, i1, jnp.where(ids == 1, i2, 0))


def _tc_moe(h, agg, w_gate_moe, W_e, b_e, w_gate_pur, W_c, b_c):
    bn = 400
    grid = (_N // bn,)
    wf = W_e.transpose(1, 0, 2).reshape(_D, _E_EXP * _D)
    bf = b_e.reshape(1, _E_EXP * _D)
    bc = b_c.reshape(1, _C)
    blk = lambda shape: pl.BlockSpec(shape, lambda i: (0, 0))
    out = pl.pallas_call(
        _tc_moe_kernel,
        grid=grid,
        in_specs=[
            pl.BlockSpec((bn, _D), lambda i: (i, 0)),
            pl.BlockSpec((bn, _D), lambda i: (i, 0)),
            blk((_D, _E_EXP * _D)),
            blk((1, _E_EXP * _D)),
            blk((_D, _E_EXP)),
            blk((_D, _E_EXP)),
            blk((_D, _C)),
            blk((1, _C)),
        ],
        out_specs=[
            pl.BlockSpec((bn, _C), lambda i: (i, 0)),
            pl.BlockSpec((bn, _E_EXP), lambda i: (i, 0)),
            pl.BlockSpec((bn, _E_EXP), lambda i: (i, 0)),
        ],
        out_shape=[
            jax.ShapeDtypeStruct((_N, _C), jnp.float32),
            jax.ShapeDtypeStruct((_N, _E_EXP), jnp.float32),
            jax.ShapeDtypeStruct((_N, _E_EXP), jnp.int32),
        ],
    )(h, agg, wf, bf, w_gate_moe, w_gate_pur, W_c, bc)
    logits, gates, idx8 = out
    return logits, gates, idx8[:, :2]


def kernel(h, edge_index, edge_weight, w_gate_moe, W_e, b_e, w_gate_pur, W_c, b_c):
    src = edge_index[0]
    dst = edge_index[1]
    agg = _sc_agg(h, src, dst, edge_weight)
    return _tc_moe(h, agg, w_gate_moe, W_e, b_e, w_gate_pur, W_c, b_c)
